# baseline (speedup 1.0000x reference)
import dataclasses
import functools

import jax
import jax.numpy as jnp
from jax import lax
from jax.experimental import pallas as pl
from jax.experimental.pallas import tpu as pltpu
from jax.experimental.pallas import tpu_sc as plsc

NC = 2
NS = 16
L = 16
NW = NC * NS
K = 48
D = 128
DE = D + L


def _edge_kernel(nch, npad, heads):
    ch_per_w = nch // NW
    rows_per_t = npad // NS
    mesh = plsc.VectorSubcoreMesh(core_axis_name="c", subcore_axis_name="s")
    cp = pltpu.CompilerParams()
    if "needs_layout_passes" in pltpu.CompilerParams.__dataclass_fields__:
        cp = dataclasses.replace(cp, needs_layout_passes=False)
    if "use_tc_tiling_on_sc" in pltpu.CompilerParams.__dataclass_fields__:
        cp = dataclasses.replace(cp, use_tc_tiling_on_sc=False)

    @functools.partial(
        pl.kernel,
        compiler_params=cp,
        out_type=jax.ShapeDtypeStruct((NC, npad, DE), jnp.float32),
        mesh=mesh,
        scratch_types=[
            pltpu.VMEM((2, K), jnp.int32),
            pltpu.VMEM((K, DE), jnp.float32),
            pltpu.VMEM((K, D), jnp.float32),
            pltpu.VMEM((2, K), jnp.int32),
            pltpu.VMEM((K, DE), jnp.float32),
            pltpu.VMEM((K, D), jnp.float32),
            pltpu.VMEM((8, L), jnp.float32),
            pltpu.VMEM_SHARED((npad, DE), jnp.float32),
            pltpu.SemaphoreType.DMA,
            pltpu.SemaphoreType.DMA,
            pltpu.SemaphoreType.DMA,
            pltpu.SemaphoreType.DMA,
        ],
    )
    def k(sd_hbm, xl_hbm, xr_hbm, att_hbm, acc_hbm,
          sdbA, xlbA, xrbA, sdbB, xlbB, xrbB,
          attb, acc, semA1, semA2, semB1, semB2):
        c = lax.axis_index("c")
        s = lax.axis_index("s")
        wid = s * NC + c
        row0 = s * rows_per_t
        bufA = (sdbA, xlbA, xrbA, semA1, semA2)
        bufB = (sdbB, xlbB, xrbB, semB1, semB2)

        @pl.loop(0, K)
        def _(i):
            for j in range(DE // L):
                xlbA[i, pl.ds(j * L, L)] = jnp.zeros((L,), jnp.float32)

        cps = []
        r0 = 0
        while r0 < rows_per_t:
            sz = min(K, rows_per_t - r0)
            cps.append(pltpu.async_copy(xlbA.at[pl.ds(0, sz)],
                                        acc.at[pl.ds(row0 + r0, sz)], semA1))
            r0 += sz
        for cp in cps:
            cp.wait()
        plsc.subcore_barrier()

        pltpu.sync_copy(att_hbm, attb)

        def start_gather(t, buf):
            sdb, xlb, xrb, sem1, sem2 = buf
            ch = wid * ch_per_w + t
            pltpu.sync_copy(sd_hbm.at[ch], sdb)
            pltpu.async_copy(xl_hbm.at[sdb.at[0]], xlb, sem1)
            pltpu.async_copy(xr_hbm.at[sdb.at[1]], xrb, sem2)

        def process(buf):
            sdb, xlb, xrb, sem1, sem2 = buf
            pltpu.make_async_copy(xl_hbm.at[sdb.at[0]], xlb, sem1).wait()
            pltpu.make_async_copy(xr_hbm.at[sdb.at[1]], xrb, sem2).wait()

            @plsc.parallel_loop(0, K, 1, unroll=4)
            def _(e):
                xls = []
                ws = []
                for j in range(D // L):
                    xl_j = xlb[e, pl.ds(j * L, L)]
                    xr_j = xrb[e, pl.ds(j * L, L)]
                    sm = xl_j + xr_j
                    m = jnp.maximum(sm, sm * 0.2)
                    xls.append(xl_j)
                    ws.append(m * attb[j, :])
                ones_j = xlb[e, pl.ds(D, L)]
                if heads == 1:
                    tot = ws[0]
                    for j in range(1, D // L):
                        tot = tot + ws[j]
                    sc = jnp.sum(tot)
                    exv = jnp.exp(jnp.full((L,), sc, jnp.float32))
                    for j in range(D // L):
                        xlb[e, pl.ds(j * L, L)] = xls[j] * exv
                    xlb[e, pl.ds(D, L)] = ones_j * exv
                else:
                    exh = []
                    for h in range(4):
                        sc = jnp.sum(ws[2 * h] + ws[2 * h + 1])
                        exh.append(jnp.exp(jnp.full((L,), sc, jnp.float32)))
                    for j in range(D // L):
                        xlb[e, pl.ds(j * L, L)] = xls[j] * exh[j // 2]
                    lane = lax.iota(jnp.int32, L)
                    a = jnp.where(lane % 2 == 0, exh[0], exh[1])
                    b = jnp.where(lane % 2 == 0, exh[2], exh[3])
                    xlb[e, pl.ds(D, L)] = ones_j * jnp.where(lane < 2, a, b)

            pltpu.sync_copy(xlb, acc.at[sdb.at[1]], add=True)

        start_gather(0, bufA)

        @pl.loop(0, ch_per_w, step=2)
        def _(t):
            start_gather(t + 1, bufB)
            process(bufA)

            @pl.when(t + 2 < ch_per_w)
            def _():
                start_gather(t + 2, bufA)

            process(bufB)

        plsc.subcore_barrier()
        r0 = 0
        while r0 < rows_per_t:
            szA = min(K, rows_per_t - r0)
            slA = pl.ds(row0 + r0, szA)
            r0 += szA
            pltpu.sync_copy(acc.at[slA], xlbA.at[pl.ds(0, szA)])
            cpo1 = pltpu.async_copy(xlbA.at[pl.ds(0, szA)],
                                    acc_hbm.at[c, slA], semA1)
            if r0 < rows_per_t:
                szB = min(K, rows_per_t - r0)
                slB = pl.ds(row0 + r0, szB)
                r0 += szB
                pltpu.sync_copy(acc.at[slB], xlbB.at[pl.ds(0, szB)])
                pltpu.async_copy(xlbB.at[pl.ds(0, szB)],
                                 acc_hbm.at[c, slB], semB1).wait()
            cpo1.wait()

    return k


def _mm2(xp, wl, wr, heads):
    npad = xp.shape[0]

    def body(x_ref, wl_ref, wr_ref, xl_ref, xr_ref):
        xv = x_ref[...]
        xl = jnp.dot(xv, wl_ref[...], preferred_element_type=jnp.float32)
        ones = jnp.ones((npad, heads), jnp.float32)
        zeros = jnp.zeros((npad, L - heads), jnp.float32)
        xl_ref[...] = jnp.concatenate([xl, ones, zeros], axis=1)
        xr_ref[...] = jnp.dot(xv, wr_ref[...],
                              preferred_element_type=jnp.float32)

    return pl.pallas_call(
        body,
        out_shape=[jax.ShapeDtypeStruct((npad, DE), jnp.float32),
                   jax.ShapeDtypeStruct((npad, D), jnp.float32)],
    )(xp, wl, wr)


def _combine(acc, heads, bias, g, b, res=None, ln=None):
    npad = acc.shape[1]
    args = [acc, bias.reshape(1, D), g.reshape(1, D), b.reshape(1, D)]
    if res is not None:
        args.append(res)
    if ln is not None:
        args.append(ln[0].reshape(1, D))
        args.append(ln[1].reshape(1, D))

    def body(*refs):
        acc_ref, bias_ref, g_ref, b_ref = refs[:4]
        i = 4
        res_ref = None
        if res is not None:
            res_ref = refs[i]
            i += 1
        ln_refs = None
        if ln is not None:
            ln_refs = (refs[i], refs[i + 1])
            i += 2
        out_ref = refs[i]

        at = acc_ref[0] + acc_ref[1]
        nm = at[:, :D]
        if heads == 1:
            div = jnp.broadcast_to(at[:, D:D + 1], (npad, D)) + 1e-16
        else:
            parts = [jnp.broadcast_to(at[:, D + h:D + h + 1], (npad, D // 4))
                     for h in range(4)]
            div = jnp.concatenate(parts, axis=1) + 1e-16
        gat = nm / div + bias_ref[...]
        scale = g_ref[...] / jnp.sqrt(1.0 + 1e-5)
        h = gat * scale + b_ref[...]
        h = jnp.maximum(h, h * 0.2)
        if res_ref is not None:
            h = h + res_ref[...]
        if ln_refs is not None:
            mu = jnp.mean(h, axis=-1, keepdims=True)
            var = jnp.mean((h - mu) ** 2, axis=-1, keepdims=True)
            h = (h - mu) / jnp.sqrt(var + 1e-5) * ln_refs[0][...] \
                + ln_refs[1][...]
        out_ref[...] = h

    return pl.pallas_call(
        body,
        out_shape=jax.ShapeDtypeStruct((npad, D), jnp.float32),
    )(*args)


def kernel(x, edge_index, params):
    p = params
    n = x.shape[0]
    e = edge_index.shape[1]
    npad = ((n + 1 + 127) // 128) * 128
    etot = e + n
    cpw = (etot + K * NW - 1) // (K * NW)
    cpw += cpw % 2
    nch = cpw * NW
    epad = nch * K

    loops = jnp.arange(n, dtype=jnp.int32)
    fill = jnp.full((epad - etot,), n, jnp.int32)
    src_i = jnp.concatenate([edge_index[0], loops, fill]).reshape(nch, 1, K)
    dst_i = jnp.concatenate([edge_index[1], loops, fill]).reshape(nch, 1, K)
    sd = jnp.concatenate([src_i, dst_i], axis=1)

    xpad = jnp.pad(x, ((0, npad - n), (0, 0)))

    ek4 = _edge_kernel(nch, npad, 4)
    ek1 = _edge_kernel(nch, npad, 1)

    xl, xr = _mm2(xpad, p['c1_Wl'], p['c1_Wr'], 4)
    att = p['c1_att'].reshape(8, L)
    acc = ek4(sd, xl, xr, att)
    h1 = _combine(acc, 4, p['c1_b'], p['bn1_g'], p['bn1_b'])

    xl, xr = _mm2(h1, p['cm_Wl'], p['cm_Wr'], 1)
    att = p['cm_att'].reshape(8, L)
    acc = ek1(sd, xl, xr, att)
    h2 = _combine(acc, 1, p['cm_b'], p['bnm_g'], p['bnm_b'], res=h1)

    xl, xr = _mm2(h2, p['c2_Wl'], p['c2_Wr'], 1)
    att = p['c2_att'].reshape(8, L)
    acc = ek1(sd, xl, xr, att)
    h3 = _combine(acc, 1, p['c2_b'], p['bn2_g'], p['bn2_b'],
                  ln=(p['ln_g'], p['ln_b']))

    return h3[:n]


# speedup vs baseline: 1.0135x; 1.0135x over previous
import dataclasses
import functools

import jax
import jax.numpy as jnp
from jax import lax
from jax.experimental import pallas as pl
from jax.experimental.pallas import tpu as pltpu
from jax.experimental.pallas import tpu_sc as plsc

NC = 2
NS = 16
L = 16
NW = NC * NS
K = 48
D = 128
DE = D + L


def _edge_kernel(nch, npad, heads):
    ch_per_w = nch // NW
    rows_per_t = npad // NS
    mesh = plsc.VectorSubcoreMesh(core_axis_name="c", subcore_axis_name="s")
    cp = pltpu.CompilerParams()
    if "needs_layout_passes" in pltpu.CompilerParams.__dataclass_fields__:
        cp = dataclasses.replace(cp, needs_layout_passes=False)
    if "use_tc_tiling_on_sc" in pltpu.CompilerParams.__dataclass_fields__:
        cp = dataclasses.replace(cp, use_tc_tiling_on_sc=False)

    @functools.partial(
        pl.kernel,
        compiler_params=cp,
        out_type=jax.ShapeDtypeStruct((NC, npad, DE), jnp.float32),
        mesh=mesh,
        scratch_types=[
            pltpu.VMEM((2, K), jnp.int32),
            pltpu.VMEM((K, DE), jnp.float32),
            pltpu.VMEM((K, D), jnp.float32),
            pltpu.VMEM((2, K), jnp.int32),
            pltpu.VMEM((K, DE), jnp.float32),
            pltpu.VMEM((K, D), jnp.float32),
            pltpu.VMEM((8, L), jnp.float32),
            pltpu.VMEM_SHARED((npad, DE), jnp.float32),
            pltpu.SemaphoreType.DMA,
            pltpu.SemaphoreType.DMA,
            pltpu.SemaphoreType.DMA,
            pltpu.SemaphoreType.DMA,
        ],
    )
    def k(sd_hbm, xl_hbm, xr_hbm, att_hbm, acc_hbm,
          sdbA, xlbA, xrbA, sdbB, xlbB, xrbB,
          attb, acc, semA1, semA2, semB1, semB2):
        c = lax.axis_index("c")
        s = lax.axis_index("s")
        wid = s * NC + c
        row0 = s * rows_per_t
        bufA = (sdbA, xlbA, xrbA, semA1, semA2)
        bufB = (sdbB, xlbB, xrbB, semB1, semB2)

        @pl.loop(0, K)
        def _(i):
            for j in range(DE // L):
                xlbA[i, pl.ds(j * L, L)] = jnp.zeros((L,), jnp.float32)

        cps = []
        r0 = 0
        while r0 < rows_per_t:
            sz = min(K, rows_per_t - r0)
            cps.append(pltpu.async_copy(xlbA.at[pl.ds(0, sz)],
                                        acc.at[pl.ds(row0 + r0, sz)], semA1))
            r0 += sz
        for cp in cps:
            cp.wait()
        plsc.subcore_barrier()

        pltpu.sync_copy(att_hbm, attb)

        def start_gather(t, buf):
            sdb, xlb, xrb, sem1, sem2 = buf
            ch = wid * ch_per_w + t
            pltpu.sync_copy(sd_hbm.at[ch], sdb)
            pltpu.async_copy(xl_hbm.at[sdb.at[0]], xlb, sem1)
            pltpu.async_copy(xr_hbm.at[sdb.at[1]], xrb, sem2)

        def process(buf):
            sdb, xlb, xrb, sem1, sem2 = buf
            pltpu.make_async_copy(xl_hbm.at[sdb.at[0]], xlb, sem1).wait()
            pltpu.make_async_copy(xr_hbm.at[sdb.at[1]], xrb, sem2).wait()

            @plsc.parallel_loop(0, K, 1, unroll=2)
            def _(e):
                xls = []
                ws = []
                for j in range(D // L):
                    xl_j = xlb[e, pl.ds(j * L, L)]
                    xr_j = xrb[e, pl.ds(j * L, L)]
                    sm = xl_j + xr_j
                    m = jnp.maximum(sm, sm * 0.2)
                    xls.append(xl_j)
                    ws.append(m * attb[j, :])
                ones_j = xlb[e, pl.ds(D, L)]
                if heads == 1:
                    tot = ws[0]
                    for j in range(1, D // L):
                        tot = tot + ws[j]
                    sc = jnp.sum(tot)
                    exv = jnp.exp(jnp.full((L,), sc, jnp.float32))
                    for j in range(D // L):
                        xlb[e, pl.ds(j * L, L)] = xls[j] * exv
                    xlb[e, pl.ds(D, L)] = ones_j * exv
                else:
                    exh = []
                    for h in range(4):
                        sc = jnp.sum(ws[2 * h] + ws[2 * h + 1])
                        exh.append(jnp.exp(jnp.full((L,), sc, jnp.float32)))
                    for j in range(D // L):
                        xlb[e, pl.ds(j * L, L)] = xls[j] * exh[j // 2]
                    lane = lax.iota(jnp.int32, L)
                    a = jnp.where(lane % 2 == 0, exh[0], exh[1])
                    b = jnp.where(lane % 2 == 0, exh[2], exh[3])
                    xlb[e, pl.ds(D, L)] = ones_j * jnp.where(lane < 2, a, b)

            pltpu.sync_copy(xlb, acc.at[sdb.at[1]], add=True)

        start_gather(0, bufA)

        @pl.loop(0, ch_per_w, step=2)
        def _(t):
            start_gather(t + 1, bufB)
            process(bufA)

            @pl.when(t + 2 < ch_per_w)
            def _():
                start_gather(t + 2, bufA)

            process(bufB)

        plsc.subcore_barrier()
        r0 = 0
        while r0 < rows_per_t:
            szA = min(K, rows_per_t - r0)
            slA = pl.ds(row0 + r0, szA)
            r0 += szA
            pltpu.sync_copy(acc.at[slA], xlbA.at[pl.ds(0, szA)])
            cpo1 = pltpu.async_copy(xlbA.at[pl.ds(0, szA)],
                                    acc_hbm.at[c, slA], semA1)
            if r0 < rows_per_t:
                szB = min(K, rows_per_t - r0)
                slB = pl.ds(row0 + r0, szB)
                r0 += szB
                pltpu.sync_copy(acc.at[slB], xlbB.at[pl.ds(0, szB)])
                pltpu.async_copy(xlbB.at[pl.ds(0, szB)],
                                 acc_hbm.at[c, slB], semB1).wait()
            cpo1.wait()

    return k


def _mm2(xp, wl, wr, heads):
    npad = xp.shape[0]

    def body(x_ref, wl_ref, wr_ref, xl_ref, xr_ref):
        xv = x_ref[...]
        xl = jnp.dot(xv, wl_ref[...], preferred_element_type=jnp.float32)
        ones = jnp.ones((npad, heads), jnp.float32)
        zeros = jnp.zeros((npad, L - heads), jnp.float32)
        xl_ref[...] = jnp.concatenate([xl, ones, zeros], axis=1)
        xr_ref[...] = jnp.dot(xv, wr_ref[...],
                              preferred_element_type=jnp.float32)

    return pl.pallas_call(
        body,
        out_shape=[jax.ShapeDtypeStruct((npad, DE), jnp.float32),
                   jax.ShapeDtypeStruct((npad, D), jnp.float32)],
    )(xp, wl, wr)


def _combine(acc, heads, bias, g, b, res=None, ln=None):
    npad = acc.shape[1]
    args = [acc, bias.reshape(1, D), g.reshape(1, D), b.reshape(1, D)]
    if res is not None:
        args.append(res)
    if ln is not None:
        args.append(ln[0].reshape(1, D))
        args.append(ln[1].reshape(1, D))

    def body(*refs):
        acc_ref, bias_ref, g_ref, b_ref = refs[:4]
        i = 4
        res_ref = None
        if res is not None:
            res_ref = refs[i]
            i += 1
        ln_refs = None
        if ln is not None:
            ln_refs = (refs[i], refs[i + 1])
            i += 2
        out_ref = refs[i]

        at = acc_ref[0] + acc_ref[1]
        nm = at[:, :D]
        if heads == 1:
            div = jnp.broadcast_to(at[:, D:D + 1], (npad, D)) + 1e-16
        else:
            parts = [jnp.broadcast_to(at[:, D + h:D + h + 1], (npad, D // 4))
                     for h in range(4)]
            div = jnp.concatenate(parts, axis=1) + 1e-16
        gat = nm / div + bias_ref[...]
        scale = g_ref[...] / jnp.sqrt(1.0 + 1e-5)
        h = gat * scale + b_ref[...]
        h = jnp.maximum(h, h * 0.2)
        if res_ref is not None:
            h = h + res_ref[...]
        if ln_refs is not None:
            mu = jnp.mean(h, axis=-1, keepdims=True)
            var = jnp.mean((h - mu) ** 2, axis=-1, keepdims=True)
            h = (h - mu) / jnp.sqrt(var + 1e-5) * ln_refs[0][...] \
                + ln_refs[1][...]
        out_ref[...] = h

    return pl.pallas_call(
        body,
        out_shape=jax.ShapeDtypeStruct((npad, D), jnp.float32),
    )(*args)


def kernel(x, edge_index, params):
    p = params
    n = x.shape[0]
    e = edge_index.shape[1]
    npad = ((n + 1 + 127) // 128) * 128
    etot = e + n
    cpw = (etot + K * NW - 1) // (K * NW)
    cpw += cpw % 2
    nch = cpw * NW
    epad = nch * K

    loops = jnp.arange(n, dtype=jnp.int32)
    fill = jnp.full((epad - etot,), n, jnp.int32)
    src_i = jnp.concatenate([edge_index[0], loops, fill]).reshape(nch, 1, K)
    dst_i = jnp.concatenate([edge_index[1], loops, fill]).reshape(nch, 1, K)
    sd = jnp.concatenate([src_i, dst_i], axis=1)

    xpad = jnp.pad(x, ((0, npad - n), (0, 0)))

    ek4 = _edge_kernel(nch, npad, 4)
    ek1 = _edge_kernel(nch, npad, 1)

    xl, xr = _mm2(xpad, p['c1_Wl'], p['c1_Wr'], 4)
    att = p['c1_att'].reshape(8, L)
    acc = ek4(sd, xl, xr, att)
    h1 = _combine(acc, 4, p['c1_b'], p['bn1_g'], p['bn1_b'])

    xl, xr = _mm2(h1, p['cm_Wl'], p['cm_Wr'], 1)
    att = p['cm_att'].reshape(8, L)
    acc = ek1(sd, xl, xr, att)
    h2 = _combine(acc, 1, p['cm_b'], p['bnm_g'], p['bnm_b'], res=h1)

    xl, xr = _mm2(h2, p['c2_Wl'], p['c2_Wr'], 1)
    att = p['c2_att'].reshape(8, L)
    acc = ek1(sd, xl, xr, att)
    h3 = _combine(acc, 1, p['c2_b'], p['bn2_g'], p['bn2_b'],
                  ln=(p['ln_g'], p['ln_b']))

    return h3[:n]


# speedup vs baseline: 1.0336x; 1.0199x over previous
import dataclasses
import functools

import jax
import jax.numpy as jnp
from jax import lax
from jax.experimental import pallas as pl
from jax.experimental.pallas import tpu as pltpu
from jax.experimental.pallas import tpu_sc as plsc

NC = 2
NS = 16
L = 16
NW = NC * NS
K = 48
D = 128
DE = D + L


def _edge_kernel(nch, npad, heads):
    ch_per_w = nch // NW
    rows_per_t = npad // NS
    mesh = plsc.VectorSubcoreMesh(core_axis_name="c", subcore_axis_name="s")
    cp = pltpu.CompilerParams()
    if "needs_layout_passes" in pltpu.CompilerParams.__dataclass_fields__:
        cp = dataclasses.replace(cp, needs_layout_passes=False)
    if "use_tc_tiling_on_sc" in pltpu.CompilerParams.__dataclass_fields__:
        cp = dataclasses.replace(cp, use_tc_tiling_on_sc=False)

    @functools.partial(
        pl.kernel,
        compiler_params=cp,
        out_type=jax.ShapeDtypeStruct((NC, npad, DE), jnp.float32),
        mesh=mesh,
        scratch_types=[
            pltpu.VMEM((2, K), jnp.int32),
            pltpu.VMEM((K, DE), jnp.float32),
            pltpu.VMEM((K, D), jnp.float32),
            pltpu.VMEM((2, K), jnp.int32),
            pltpu.VMEM((K, DE), jnp.float32),
            pltpu.VMEM((K, D), jnp.float32),
            pltpu.VMEM((8, L), jnp.float32),
            pltpu.VMEM_SHARED((npad, DE), jnp.float32),
            pltpu.SemaphoreType.DMA,
            pltpu.SemaphoreType.DMA,
            pltpu.SemaphoreType.DMA,
            pltpu.SemaphoreType.DMA,
        ],
    )
    def k(sd_hbm, xl_hbm, xr_hbm, att_hbm, acc_hbm,
          sdbA, xlbA, xrbA, sdbB, xlbB, xrbB,
          attb, acc, semA1, semA2, semB1, semB2):
        c = lax.axis_index("c")
        s = lax.axis_index("s")
        wid = s * NC + c
        row0 = s * rows_per_t
        bufA = (sdbA, xlbA, xrbA, semA1, semA2)
        bufB = (sdbB, xlbB, xrbB, semB1, semB2)

        @pl.loop(0, K)
        def _(i):
            for j in range(DE // L):
                xlbA[i, pl.ds(j * L, L)] = jnp.zeros((L,), jnp.float32)

        cps = []
        r0 = 0
        while r0 < rows_per_t:
            sz = min(K, rows_per_t - r0)
            cps.append(pltpu.async_copy(xlbA.at[pl.ds(0, sz)],
                                        acc.at[pl.ds(row0 + r0, sz)], semA1))
            r0 += sz
        for cp in cps:
            cp.wait()
        plsc.subcore_barrier()

        pltpu.sync_copy(att_hbm, attb)

        def start_gather(t, buf):
            sdb, xlb, xrb, sem1, sem2 = buf
            ch = wid * ch_per_w + t
            pltpu.sync_copy(sd_hbm.at[ch], sdb)
            pltpu.async_copy(xl_hbm.at[sdb.at[0]], xlb, sem1)
            pltpu.async_copy(xr_hbm.at[sdb.at[1]], xrb, sem2)

        def process(buf):
            sdb, xlb, xrb, sem1, sem2 = buf
            pltpu.make_async_copy(xl_hbm.at[sdb.at[0]], xlb, sem1).wait()
            pltpu.make_async_copy(xr_hbm.at[sdb.at[1]], xrb, sem2).wait()

            @plsc.parallel_loop(0, K, 1, unroll=2)
            def _(e):
                xls = []
                ws = []
                for j in range(D // L):
                    xl_j = xlb[e, pl.ds(j * L, L)]
                    xr_j = xrb[e, pl.ds(j * L, L)]
                    sm = xl_j + xr_j
                    m = jnp.maximum(sm, sm * 0.2)
                    xls.append(xl_j)
                    ws.append(m * attb[j, :])
                ones_j = xlb[e, pl.ds(D, L)]
                if heads == 1:
                    tot = ws[0]
                    for j in range(1, D // L):
                        tot = tot + ws[j]
                    sc = jnp.sum(tot)
                    exv = jnp.exp(jnp.full((L,), sc, jnp.float32))
                    for j in range(D // L):
                        xlb[e, pl.ds(j * L, L)] = xls[j] * exv
                    xlb[e, pl.ds(D, L)] = ones_j * exv
                else:
                    exh = []
                    for h in range(4):
                        sc = jnp.sum(ws[2 * h] + ws[2 * h + 1])
                        exh.append(jnp.exp(jnp.full((L,), sc, jnp.float32)))
                    for j in range(D // L):
                        xlb[e, pl.ds(j * L, L)] = xls[j] * exh[j // 2]
                    lane = lax.iota(jnp.int32, L)
                    a = jnp.where(lane % 2 == 0, exh[0], exh[1])
                    b = jnp.where(lane % 2 == 0, exh[2], exh[3])
                    xlb[e, pl.ds(D, L)] = ones_j * jnp.where(lane < 2, a, b)

            pltpu.sync_copy(xlb, acc.at[sdb.at[1]], add=True)

        start_gather(0, bufA)

        @pl.loop(0, ch_per_w, step=2)
        def _(t):
            start_gather(t + 1, bufB)
            process(bufA)

            @pl.when(t + 2 < ch_per_w)
            def _():
                start_gather(t + 2, bufA)

            process(bufB)

        plsc.subcore_barrier()
        r0 = 0
        while r0 < rows_per_t:
            szA = min(K, rows_per_t - r0)
            slA = pl.ds(row0 + r0, szA)
            r0 += szA
            pltpu.sync_copy(acc.at[slA], xlbA.at[pl.ds(0, szA)])
            cpo1 = pltpu.async_copy(xlbA.at[pl.ds(0, szA)],
                                    acc_hbm.at[c, slA], semA1)
            if r0 < rows_per_t:
                szB = min(K, rows_per_t - r0)
                slB = pl.ds(row0 + r0, szB)
                r0 += szB
                pltpu.sync_copy(acc.at[slB], xlbB.at[pl.ds(0, szB)])
                pltpu.async_copy(xlbB.at[pl.ds(0, szB)],
                                 acc_hbm.at[c, slB], semB1).wait()
            cpo1.wait()

    return k


def _mm2(xp, wl, wr, heads):
    npad = xp.shape[0]

    def body(x_ref, wl_ref, wr_ref, xl_ref, xr_ref):
        xv = x_ref[...]
        xl = jnp.dot(xv, wl_ref[...], preferred_element_type=jnp.float32)
        ones = jnp.ones((npad, heads), jnp.float32)
        zeros = jnp.zeros((npad, L - heads), jnp.float32)
        xl_ref[...] = jnp.concatenate([xl, ones, zeros], axis=1)
        xr_ref[...] = jnp.dot(xv, wr_ref[...],
                              preferred_element_type=jnp.float32)

    return pl.pallas_call(
        body,
        out_shape=[jax.ShapeDtypeStruct((npad, DE), jnp.float32),
                   jax.ShapeDtypeStruct((npad, D), jnp.float32)],
    )(xp, wl, wr)


def _combine(acc, heads, bias, g, b, res=None, ln=None, mm=None):
    npad = acc.shape[1]
    args = [acc, bias.reshape(1, D), g.reshape(1, D), b.reshape(1, D)]
    if res is not None:
        args.append(res)
    if ln is not None:
        args.append(ln[0].reshape(1, D))
        args.append(ln[1].reshape(1, D))
    if mm is not None:
        args.append(mm[0])
        args.append(mm[1])

    def body(*refs):
        acc_ref, bias_ref, g_ref, b_ref = refs[:4]
        i = 4
        res_ref = None
        if res is not None:
            res_ref = refs[i]
            i += 1
        ln_refs = None
        if ln is not None:
            ln_refs = (refs[i], refs[i + 1])
            i += 2
        mm_refs = None
        if mm is not None:
            mm_refs = (refs[i], refs[i + 1])
            i += 2
        out_ref = refs[i]
        if mm is not None:
            xl_ref = refs[i + 1]
            xr_ref = refs[i + 2]

        at = acc_ref[0] + acc_ref[1]
        nm = at[:, :D]
        if heads == 1:
            div = jnp.broadcast_to(at[:, D:D + 1], (npad, D)) + 1e-16
        else:
            parts = [jnp.broadcast_to(at[:, D + h:D + h + 1], (npad, D // 4))
                     for h in range(4)]
            div = jnp.concatenate(parts, axis=1) + 1e-16
        gat = nm / div + bias_ref[...]
        scale = g_ref[...] / jnp.sqrt(1.0 + 1e-5)
        h = gat * scale + b_ref[...]
        h = jnp.maximum(h, h * 0.2)
        if res_ref is not None:
            h = h + res_ref[...]
        if ln_refs is not None:
            mu = jnp.mean(h, axis=-1, keepdims=True)
            var = jnp.mean((h - mu) ** 2, axis=-1, keepdims=True)
            h = (h - mu) / jnp.sqrt(var + 1e-5) * ln_refs[0][...] \
                + ln_refs[1][...]
        out_ref[...] = h
        if mm is not None:
            heads_next = mm[2]
            xl = jnp.dot(h, mm_refs[0][...],
                         preferred_element_type=jnp.float32)
            ones = jnp.ones((npad, heads_next), jnp.float32)
            zeros = jnp.zeros((npad, L - heads_next), jnp.float32)
            xl_ref[...] = jnp.concatenate([xl, ones, zeros], axis=1)
            xr_ref[...] = jnp.dot(h, mm_refs[1][...],
                                  preferred_element_type=jnp.float32)

    out_shape = jax.ShapeDtypeStruct((npad, D), jnp.float32)
    if mm is not None:
        out_shape = [out_shape,
                     jax.ShapeDtypeStruct((npad, DE), jnp.float32),
                     jax.ShapeDtypeStruct((npad, D), jnp.float32)]
    return pl.pallas_call(body, out_shape=out_shape)(*args)


def kernel(x, edge_index, params):
    p = params
    n = x.shape[0]
    e = edge_index.shape[1]
    npad = ((n + 1 + 127) // 128) * 128
    etot = e + n
    cpw = (etot + K * NW - 1) // (K * NW)
    cpw += cpw % 2
    nch = cpw * NW
    epad = nch * K

    loops = jnp.arange(n, dtype=jnp.int32)
    fill = jnp.full((epad - etot,), n, jnp.int32)
    src_i = jnp.concatenate([edge_index[0], loops, fill]).reshape(nch, 1, K)
    dst_i = jnp.concatenate([edge_index[1], loops, fill]).reshape(nch, 1, K)
    sd = jnp.concatenate([src_i, dst_i], axis=1)

    xpad = jnp.pad(x, ((0, npad - n), (0, 0)))

    ek4 = _edge_kernel(nch, npad, 4)
    ek1 = _edge_kernel(nch, npad, 1)

    xl, xr = _mm2(xpad, p['c1_Wl'], p['c1_Wr'], 4)
    att = p['c1_att'].reshape(8, L)
    acc = ek4(sd, xl, xr, att)
    h1, xl, xr = _combine(acc, 4, p['c1_b'], p['bn1_g'], p['bn1_b'],
                          mm=(p['cm_Wl'], p['cm_Wr'], 1))

    att = p['cm_att'].reshape(8, L)
    acc = ek1(sd, xl, xr, att)
    h2, xl, xr = _combine(acc, 1, p['cm_b'], p['bnm_g'], p['bnm_b'], res=h1,
                          mm=(p['c2_Wl'], p['c2_Wr'], 1))

    att = p['c2_att'].reshape(8, L)
    acc = ek1(sd, xl, xr, att)
    h3 = _combine(acc, 1, p['c2_b'], p['bn2_g'], p['bn2_b'],
                  ln=(p['ln_g'], p['ln_b']))

    return h3[:n]
